# baseline (speedup 1.0000x reference)
import functools

import jax
import jax.numpy as jnp
from jax import lax
from jax.experimental import pallas as pl
from jax.experimental.pallas import tpu as pltpu
from jax.experimental.pallas import tpu_sc as plsc

_NC = 2
_NS = 16
_NW = _NC * _NS
_L = 16


def _round_up(v, m):
    return (v + m - 1) // m * m


def _proj_body(x_ref, w_ref, bias_ref, o_ref):
    o_ref[...] = (
        jnp.dot(x_ref[...], w_ref[...], preferred_element_type=jnp.float32)
        + bias_ref[...]
    )


def _soft_body(degp_ref, h_ref, sp_ref, split_ref, bg_ref, nm_ref, ns_ref):
    npad = h_ref.shape[0]
    nb = split_ref.shape[0]
    deg = degp_ref[0, :] + degp_ref[1, :] + 1.0
    dis = lax.rsqrt(deg)
    g = dis * h_ref[...]
    s = sp_ref[0, :] + sp_ref[1, :]
    natt = dis * (s + g) + bg_ref[...]

    spl = split_ref[...]
    ib = lax.broadcasted_iota(jnp.int32, (nb, nb), 0)
    jb = lax.broadcasted_iota(jnp.int32, (nb, nb), 1)
    oincl = jnp.sum(jnp.where(jb <= ib, spl[None, :], 0), axis=1)
    oexcl = oincl - spl
    ii = lax.broadcasted_iota(jnp.int32, (nb, npad), 1)
    mask = (ii >= oexcl[:, None]) & (ii < oincl[:, None])

    def segsoft(v):
        m = jnp.max(jnp.where(mask, v[None, :], -jnp.inf), axis=1)
        mn = jnp.sum(jnp.where(mask, m[:, None], 0.0), axis=0)
        e = jnp.exp(v - mn)
        sb = jnp.sum(jnp.where(mask, e[None, :], 0.0), axis=1)
        sn = jnp.sum(jnp.where(mask, sb[:, None], 0.0), axis=0)
        return e / jnp.maximum(sn, 1e-16)

    nm = segsoft(natt)
    nm_ref[...] = nm
    ns_ref[...] = segsoft(1.0 - nm)


def _make_deg_kernel(ep, ew, npad):
    vecs = ew // _L
    nvec = npad // _L
    mesh = plsc.VectorSubcoreMesh(core_axis_name="c", subcore_axis_name="s")

    @functools.partial(
        pl.kernel,
        out_type=[jax.ShapeDtypeStruct((_NC, npad), jnp.float32)],
        mesh=mesh,
        scratch_types=[
            pltpu.VMEM((ew,), jnp.int32),
            pltpu.VMEM((ew,), jnp.float32),
            pltpu.VMEM((npad,), jnp.float32),
            pltpu.VMEM_SHARED((npad,), jnp.float32),
            pltpu.SemaphoreType.DMA,
            pltpu.SemaphoreType.DMA,
        ],
        compiler_params=pltpu.CompilerParams(needs_layout_passes=False),
    )
    def dk(row_h, degp_h, row_v, one_v, zero_v, acc, sem_r, sem_sc):
        c = lax.axis_index("c")
        s = lax.axis_index("s")
        w = s * _NC + c
        base = w * ew
        dr = pltpu.async_copy(row_h.at[pl.ds(base, ew)], row_v, sem_r)
        ones = jnp.full((_L,), 1.0, jnp.float32)
        zeros = jnp.zeros((_L,), jnp.float32)

        @plsc.parallel_loop(0, vecs, unroll=8)
        def _(i):
            one_v[pl.ds(i * _L, _L)] = ones

        @pl.when(s == 0)
        def _():
            @plsc.parallel_loop(0, nvec, unroll=8)
            def _(i):
                zero_v[pl.ds(i * _L, _L)] = zeros

            pltpu.sync_copy(zero_v, acc)

        plsc.subcore_barrier()
        dr.wait()
        pltpu.async_copy(one_v, acc.at[row_v], sem_sc, add=True).wait()
        plsc.subcore_barrier()

        @pl.when(s == 0)
        def _():
            pltpu.sync_copy(acc, degp_h.at[c])

    return dk


def _make_edge_kernel(ep, ew, npad):
    vecs = ew // _L
    nvec = npad // _L
    nsl = npad // _NS
    nslv = nsl // _L
    mesh = plsc.VectorSubcoreMesh(core_axis_name="c", subcore_axis_name="s")

    @functools.partial(
        pl.kernel,
        out_type=[
            jax.ShapeDtypeStruct((ep,), jnp.float32),
            jax.ShapeDtypeStruct((ep,), jnp.float32),
            jax.ShapeDtypeStruct((_NC, npad), jnp.float32),
        ],
        mesh=mesh,
        scratch_types=[
            pltpu.VMEM((ew,), jnp.int32),
            pltpu.VMEM((ew,), jnp.int32),
            pltpu.VMEM((npad,), jnp.float32),
            pltpu.VMEM((npad,), jnp.float32),
            pltpu.VMEM((npad,), jnp.float32),
            pltpu.VMEM((ew,), jnp.float32),
            pltpu.VMEM((ew,), jnp.float32),
            pltpu.VMEM((ew,), jnp.float32),
            pltpu.VMEM((npad,), jnp.float32),
            pltpu.VMEM((nsl,), jnp.float32),
            pltpu.VMEM((nsl,), jnp.float32),
            pltpu.VMEM((nsl,), jnp.float32),
            pltpu.VMEM((nsl,), jnp.float32),
            pltpu.VMEM_SHARED((npad,), jnp.float32),
            pltpu.VMEM_SHARED((npad,), jnp.float32),
            pltpu.SemaphoreType.DMA,
            pltpu.SemaphoreType.DMA,
            pltpu.SemaphoreType.DMA,
            pltpu.SemaphoreType.DMA,
            pltpu.SemaphoreType.DMA,
            pltpu.SemaphoreType.DMA,
            pltpu.SemaphoreType.DMA,
            pltpu.SemaphoreType.DMA,
            pltpu.SemaphoreType.DMA,
            pltpu.SemaphoreType.DMA,
        ],
        compiler_params=pltpu.CompilerParams(needs_layout_passes=False),
    )
    def ek(row_h, col_h, a_h, b_h, h_h, degp_h, eam_h, eas_h, sp_h,
           row_v, col_v, a_v, b_v, g_v, m_v, s_v, val_v, zero_v,
           d0_v, d1_v, h_sl, g_sl, acc, g_sh,
           sem_r, sem_c, sem_a, sem_b, sem_h, sem_d0, sem_d1,
           sem_sc, sem_m, sem_s):
        c = lax.axis_index("c")
        s = lax.axis_index("s")
        w = s * _NC + c
        base = w * ew
        nbase = s * nsl
        dr = pltpu.async_copy(row_h.at[pl.ds(base, ew)], row_v, sem_r)
        dc = pltpu.async_copy(col_h.at[pl.ds(base, ew)], col_v, sem_c)
        da = pltpu.async_copy(a_h, a_v, sem_a)
        db = pltpu.async_copy(b_h, b_v, sem_b)
        dh = pltpu.async_copy(h_h.at[pl.ds(nbase, nsl)], h_sl, sem_h)
        dd0 = pltpu.async_copy(degp_h.at[0].at[pl.ds(nbase, nsl)], d0_v, sem_d0)
        dd1 = pltpu.async_copy(degp_h.at[1].at[pl.ds(nbase, nsl)], d1_v, sem_d1)
        zeros = jnp.zeros((_L,), jnp.float32)

        @pl.when(s == 0)
        def _():
            @plsc.parallel_loop(0, nvec, unroll=8)
            def _(i):
                zero_v[pl.ds(i * _L, _L)] = zeros

            pltpu.sync_copy(zero_v, acc)

        dh.wait()
        dd0.wait()
        dd1.wait()
        magic = jnp.full((_L,), 0x5F3759DF, jnp.int32)

        @plsc.parallel_loop(0, nslv, unroll=4)
        def _(i):
            sl = pl.ds(i * _L, _L)
            deg = d0_v[sl] + d1_v[sl] + 1.0
            y = plsc.bitcast(
                magic - lax.shift_right_logical(plsc.bitcast(deg, jnp.int32), 1),
                jnp.float32,
            )
            hd = -0.5 * deg
            y = y * (1.5 + hd * y * y)
            y = y * (1.5 + hd * y * y)
            y = y * (1.5 + hd * y * y)
            g_sl[sl] = y * h_sl[sl]

        pltpu.sync_copy(g_sl, g_sh.at[pl.ds(nbase, nsl)])
        plsc.subcore_barrier()
        pltpu.sync_copy(g_sh, g_v)
        dr.wait()
        dc.wait()
        da.wait()
        db.wait()

        @plsc.parallel_loop(0, vecs, unroll=4)
        def _(i):
            sl = pl.ds(i * _L, _L)
            r = row_v[sl]
            cc = col_v[sl]
            av = plsc.load_gather(a_v, [r])
            bv = plsc.load_gather(b_v, [cc])
            m = 1.0 / (1.0 + jnp.exp(-(av + bv)))
            m_v[sl] = m
            s_v[sl] = 1.0 - m
            val_v[sl] = plsc.load_gather(g_v, [r])

        dsc = pltpu.async_copy(val_v, acc.at[col_v], sem_sc, add=True)
        dm = pltpu.async_copy(m_v, eam_h.at[pl.ds(base, ew)], sem_m)
        ds2 = pltpu.async_copy(s_v, eas_h.at[pl.ds(base, ew)], sem_s)
        dsc.wait()
        dm.wait()
        ds2.wait()
        plsc.subcore_barrier()

        @pl.when(s == 0)
        def _():
            pltpu.sync_copy(acc, sp_h.at[c])

    return ek


def kernel(x, edge_index, split_n, We, be, Wg, bg):
    n, d = x.shape
    e = edge_index.shape[1]
    npad = _round_up(n, _NS * 128)
    ew = _round_up(e, _NW * _L) // _NW
    ep = ew * _NW

    xp = jnp.pad(x, ((0, npad - n), (0, 0)))
    w3 = jnp.concatenate([We[:d], We[d:], Wg], axis=1)
    bias = jnp.stack([be[0], jnp.float32(0.0), jnp.float32(0.0)])[None, :]

    proj = pl.pallas_call(
        _proj_body,
        out_shape=jax.ShapeDtypeStruct((npad, 3), jnp.float32),
    )(xp, w3, bias)
    a = proj[:, 0]
    b = proj[:, 1]
    h = proj[:, 2]

    pad_e = jnp.full((ep - e,), n, dtype=jnp.int32)
    rowp = jnp.concatenate([edge_index[0], pad_e])
    colp = jnp.concatenate([edge_index[1], pad_e])

    (degp,) = _make_deg_kernel(ep, ew, npad)(rowp)

    eam, eas, sp = _make_edge_kernel(ep, ew, npad)(
        rowp, colp, a, b, h, degp
    )

    nm, ns = pl.pallas_call(
        _soft_body,
        out_shape=[
            jax.ShapeDtypeStruct((npad,), jnp.float32),
            jax.ShapeDtypeStruct((npad,), jnp.float32),
        ],
    )(degp, h, sp, split_n, bg)

    return (eam[:e, None], eas[:e, None], nm[:n, None], ns[:n, None])
